# baseline (speedup 1.0000x reference)
import functools

import jax
import jax.numpy as jnp
from jax import lax
from jax.experimental import pallas as pl
from jax.experimental.pallas import tpu as pltpu
from jax.experimental.pallas import tpu_sc as plsc

N = 10000
NP = 10240
E = 320000
NC = 2
NS = 16
D = 256

QR = NP // 4
SLC2 = QR // NS

B2 = 128
CH = 16
NBMAX = 160
CAPS = NBMAX * B2

NB_DEG = 79
BD = 128
E_PAD_DEG = NC * NS * NB_DEG * BD

SLC = NP // NS

RB = 1024
NRB = NP // RB

_mesh = plsc.VectorSubcoreMesh(
    core_axis_name="c", subcore_axis_name="s", num_cores=NC, num_subcores=NS)


@functools.partial(
    pl.kernel,
    out_type=(jax.ShapeDtypeStruct((NP,), jnp.float32),
              jax.ShapeDtypeStruct((NP,), jnp.float32)),
    mesh=_mesh,
    scratch_types=[
        pltpu.VMEM((NB_DEG, BD), jnp.int32),
        pltpu.VMEM((BD,), jnp.float32),
        pltpu.VMEM_SHARED((NP,), jnp.float32),
    ],
)
def _deg_kernel(dst_hbm, zeros_hbm, deg0_out, deg1_out, dst_v, ones_v, acc):
    c = lax.axis_index("c")
    s = lax.axis_index("s")
    for i in range(BD // 16):
        ones_v[pl.ds(i * 16, 16)] = jnp.ones((16,), jnp.float32)
    pltpu.sync_copy(dst_hbm.at[c, s], dst_v)
    pltpu.sync_copy(zeros_hbm.at[pl.ds(s * SLC, SLC)],
                    acc.at[pl.ds(s * SLC, SLC)])
    plsc.subcore_barrier()

    def body(j, carry):
        pltpu.sync_copy(ones_v, acc.at[dst_v.at[j]], add=True)
        return carry

    lax.fori_loop(0, NB_DEG, body, 0)
    plsc.subcore_barrier()

    @pl.when(c == 0)
    def _():
        pltpu.sync_copy(acc.at[pl.ds(s * SLC, SLC)],
                        deg0_out.at[pl.ds(s * SLC, SLC)])

    @pl.when(c == 1)
    def _():
        pltpu.sync_copy(acc.at[pl.ds(s * SLC, SLC)],
                        deg1_out.at[pl.ds(s * SLC, SLC)])


@functools.partial(
    pl.kernel,
    out_type=jax.ShapeDtypeStruct((NC, QR, 2, 128), jnp.float32),
    mesh=_mesh,
    scratch_types=[
        pltpu.VMEM((128,), jnp.int32),
        pltpu.VMEM((CH, B2), jnp.int32),
        pltpu.VMEM((CH, B2), jnp.int32),
        pltpu.VMEM((B2, 2, 128), jnp.float32),
        pltpu.VMEM((B2, 2, 128), jnp.float32),
        pltpu.VMEM_SHARED((QR, 2, 128), jnp.float32),
        pltpu.SemaphoreType.DMA,
        pltpu.SemaphoreType.DMA,
        pltpu.SemaphoreType.DMA,
        pltpu.SemaphoreType.DMA,
    ],
)
def _layer_sc(h_hbm, h2_hbm, src_hbm, dst_hbm, nch_hbm, out_hbm, nch_v,
              src_v, dst_v, rows0, rows1, acc, gs0, gs1, ss0, ss1):
    c = lax.axis_index("c")
    s = lax.axis_index("s")
    pltpu.sync_copy(nch_hbm.at[pl.ds(c * 128, 128)], nch_v)
    mine = nch_v[pl.ds(s, 16)][0]
    base = pl.multiple_of(nch_v[pl.ds(16, 16)][0], QR)

    pltpu.sync_copy(h2_hbm.at[pl.ds(base + s * SLC2, SLC2)],
                    acc.at[pl.ds(s * SLC2, SLC2)])
    plsc.subcore_barrier()

    bufs = (rows0, rows1)
    gsems = (gs0, gs1)
    ssems = (ss0, ss1)

    def chunk(k, carry):
        pltpu.sync_copy(src_hbm.at[c, s, pl.ds(k * CH, CH)], src_v)
        pltpu.sync_copy(dst_hbm.at[c, s, pl.ds(k * CH, CH)], dst_v)
        gdesc = [None, None]
        sdesc = [None, None]
        gdesc[0] = pltpu.async_copy(h_hbm.at[src_v.at[0]], rows0, gsems[0])
        for j in range(CH):
            cur, nxt = j % 2, (j + 1) % 2
            if j + 1 < CH:
                if sdesc[nxt] is not None:
                    sdesc[nxt].wait()
                gdesc[nxt] = pltpu.async_copy(
                    h_hbm.at[src_v.at[j + 1]], bufs[nxt], gsems[nxt])
            gdesc[cur].wait()
            sdesc[cur] = pltpu.async_copy(
                bufs[cur], acc.at[dst_v.at[j]], ssems[cur], add=True)
        sdesc[0].wait()
        sdesc[1].wait()
        return carry

    lax.fori_loop(0, mine, chunk, 0)
    plsc.subcore_barrier()
    pltpu.sync_copy(acc.at[pl.ds(s * SLC2, SLC2)],
                    out_hbm.at[c, pl.ds(s * SLC2, SLC2)])


def _dinv_from_deg(deg2):
    def body(d_ref, o_ref):
        o_ref[...] = lax.rsqrt(d_ref[0] + d_ref[1] + 1.0)

    return pl.pallas_call(
        body,
        in_specs=[pl.BlockSpec((2, NP // 128, 128), lambda: (0, 0, 0))],
        out_specs=pl.BlockSpec((NP // 128, 128), lambda: (0, 0)),
        out_shape=jax.ShapeDtypeStruct((NP // 128, 128), jnp.float32),
    )(deg2)


def _mm_first(x, w1, dinv):
    def body(x_ref, w_ref, d_ref, o_ref):
        o_ref[...] = d_ref[...] * jnp.dot(
            x_ref[...], w_ref[...], preferred_element_type=jnp.float32)

    return pl.pallas_call(
        body,
        grid=(NRB,),
        in_specs=[
            pl.BlockSpec((RB, 128), lambda r: (r, 0)),
            pl.BlockSpec((128, D), lambda r: (0, 0)),
            pl.BlockSpec((RB, 1), lambda r: (r, 0)),
        ],
        out_specs=pl.BlockSpec((RB, D), lambda r: (r, 0)),
        out_shape=jax.ShapeDtypeStruct((NP, D), jnp.float32),
    )(x, w1, dinv)


def _mm_mid(agg, bprev, dinv, w):
    def body(a_ref, b_ref, d_ref, w_ref, o_ref):
        d = d_ref[...]
        z = jnp.maximum(d * a_ref[...] + b_ref[0:1, :], 0.0)
        o_ref[...] = d * jnp.dot(z, w_ref[...],
                                 preferred_element_type=jnp.float32)

    return pl.pallas_call(
        body,
        grid=(NRB,),
        in_specs=[
            pl.BlockSpec((RB, D), lambda r: (r, 0)),
            pl.BlockSpec((8, D), lambda r: (0, 0)),
            pl.BlockSpec((RB, 1), lambda r: (r, 0)),
            pl.BlockSpec((D, D), lambda r: (0, 0)),
        ],
        out_specs=pl.BlockSpec((RB, D), lambda r: (r, 0)),
        out_shape=jax.ShapeDtypeStruct((NP, D), jnp.float32),
    )(agg, bprev, dinv, w)


def _mm_final(agg, b3p, dinv):
    def body(a_ref, b_ref, d_ref, o_ref):
        o_ref[...] = d_ref[...] * a_ref[...] + b_ref[0:1, :]

    return pl.pallas_call(
        body,
        grid=(NRB,),
        in_specs=[
            pl.BlockSpec((RB, 128), lambda r: (r, 0)),
            pl.BlockSpec((8, 128), lambda r: (0, 0)),
            pl.BlockSpec((RB, 1), lambda r: (r, 0)),
        ],
        out_specs=pl.BlockSpec((RB, 128), lambda r: (r, 0)),
        out_shape=jax.ShapeDtypeStruct((NP, 128), jnp.float32),
    )(agg, b3p, dinv)


def _pad_bias(b):
    return jnp.zeros((8, b.shape[0]), jnp.float32).at[0].set(b)


def kernel(x, edge_index, W1, b1, W2, b2, W3, b3):
    src = edge_index[0]
    dst = edge_index[1]

    dstd = jnp.concatenate(
        [dst, jnp.full((E_PAD_DEG - E,), N, jnp.int32)]
    ).reshape(NC, NS, NB_DEG, BD)
    deg0, deg1 = _deg_kernel(dstd, jnp.zeros((NP,), jnp.float32))
    dinv_g = _dinv_from_deg(
        jnp.stack([deg0, deg1]).reshape(2, NP // 128, 128))
    dinv = dinv_g.reshape(NP, 1)
    dinv = jnp.where(jnp.arange(NP, dtype=jnp.int32)[:, None] < N, dinv, 0.0)

    q = dst // QR
    r = jnp.zeros((E,), jnp.int32)
    cnt4 = []
    for i in range(4):
        m = (q == i).astype(jnp.int32)
        r = jnp.where(q == i, jnp.cumsum(m) - 1, r)
        cnt4.append(jnp.sum(m))
    slot = q * (NS * CAPS) + (r % NS) * CAPS + r // NS
    srcbuf = jnp.full((4 * NS * CAPS,), N, jnp.int32).at[slot].set(src)
    src5 = srcbuf.reshape(4, NS, NBMAX, B2)
    dstbuf = jnp.zeros((4 * NS * CAPS,), jnp.int32).at[slot].set(dst - q * QR)
    dst5 = dstbuf.reshape(4, NS, NBMAX, B2)

    svec = jnp.arange(NS, dtype=jnp.int32)
    cnt = jnp.stack([(cnt4[i] + NS - 1 - svec) // NS for i in range(4)])
    nch = -(-cnt // (B2 * CH))
    nch_call = []
    for k in range(2):
        nf = (jnp.zeros((NC * 128,), jnp.int32)
              .at[0:NS].set(nch[2 * k])
              .at[128:128 + NS].set(nch[2 * k + 1])
              .at[16].set(2 * k * QR)
              .at[128 + 16].set((2 * k + 1) * QR))
        nch_call.append(nf)

    b1p, b2p = _pad_bias(b1), _pad_bias(b2)
    b3p = _pad_bias(b3)
    w3p = jnp.concatenate([W3, jnp.zeros((D, 128), jnp.float32)], axis=1)
    x_pad = jnp.zeros((NP, 128), jnp.float32).at[:N].set(x)

    h1 = _mm_first(x_pad, W1, dinv)
    a1 = _agg(h1, src5, dst5, nch_call)
    h2 = _mm_mid(a1, b1p, dinv, W2)
    a2 = _agg(h2, src5, dst5, nch_call)
    h3 = _mm_mid(a2, b2p, dinv, w3p)
    a3 = _agg(h3, src5, dst5, nch_call)
    return _mm_final(a3, b3p, dinv)[:N]


def _agg(h, src5, dst5, nch_call):
    h3v = h.reshape(NP, 2, 128)
    lo = _layer_sc(h3v, h3v, src5[0:2], dst5[0:2], nch_call[0])
    hi = _layer_sc(h3v, h3v, src5[2:4], dst5[2:4], nch_call[1])
    return jnp.concatenate([lo.reshape(NP // 2, D),
                            hi.reshape(NP // 2, D)], axis=0)
